# Initial kernel scaffold; baseline (speedup 1.0000x reference)
import jax
import jax.numpy as jnp
from jax.experimental import pallas as pl


def kernel(x, low, high):
    raise NotImplementedError("write your pallas kernel here")


# speedup vs baseline: 2.6928x; 2.6928x over previous
import functools

import jax
import jax.numpy as jnp
from jax import lax
from jax.experimental import pallas as pl
from jax.experimental.pallas import tpu as pltpu
from jax.experimental.pallas import tpu_sc as plsc

_N = 64 * 8192
_NT = 16
_CHUNK = _N // _NT
_VECS = _CHUNK // 16
_UNROLL = 16

_P_LOW = 0.05
_P_HIGH = 0.95
_DECAY = 0.99
_MIN = 1.0

_KA = int(_P_LOW * (_N - 1))
_KB = int(_P_HIGH * (_N - 1))
_FRACA = _P_LOW * (_N - 1) - _KA
_FRACB = _P_HIGH * (_N - 1) - _KB

_SIGN = -2147483648
_UMAX = 0xFFFFFFFF

_mesh = plsc.VectorSubcoreMesh(
    core_axis_name="c", subcore_axis_name="s", num_cores=1, num_subcores=_NT)


def _lanesum_u32(v):
    s = v[0]
    for i in range(1, 16):
        s = s + v[i]
    return s


def _lanemin_u32(v):
    s = v[0]
    for i in range(1, 16):
        s = jnp.minimum(s, v[i])
    return s


def _body(x_hbm, p_hbm, out_hbm,
          xbuf, ubuf, pub, rd, fin, rdf, pbuf, obuf, sh_cnt, sh_fin):
    wid = lax.axis_index("s")
    base = wid * _CHUNK
    pltpu.sync_copy(x_hbm.at[pl.ds(base, _CHUNK)], xbuf)
    pltpu.sync_copy(p_hbm, pbuf)

    one = jnp.ones((16,), jnp.uint32)
    zero = jnp.zeros((16,), jnp.uint32)
    umax_v = jnp.full((16,), _UMAX, jnp.uint32)

    def map_body(i, _):
        for j in range(_UNROLL):
            off = (i * _UNROLL + j) * 16
            b = lax.bitcast_convert_type(xbuf[pl.ds(off, 16)], jnp.int32)
            u = b ^ ((b >> 31) | jnp.int32(_SIGN))
            ubuf[pl.ds(off, 16)] = lax.bitcast_convert_type(u, jnp.uint32)
        return 0
    lax.fori_loop(0, _VECS // _UNROLL, map_body, 0)

    kA1 = jnp.uint32(_KA + 1)
    kB1 = jnp.uint32(_KB + 1)

    def round_body(t, carry):
        resA, resB, bit = carry
        midA = resA | bit
        midB = resB | bit

        def cnt_body(i, acc):
            aA, aB = acc
            for j in range(_UNROLL):
                off = (i * _UNROLL + j) * 16
                u = ubuf[pl.ds(off, 16)]
                aA = aA + jnp.where(u < midA, one, zero)
                aB = aB + jnp.where(u < midB, one, zero)
            return (aA, aB)
        accA, accB = lax.fori_loop(0, _VECS // _UNROLL, cnt_body,
                                   (zero, zero))

        pub[pl.ds(0, 16)] = accA
        pub[pl.ds(16, 16)] = accB
        par = t & 1
        pltpu.sync_copy(pub, sh_cnt.at[pl.ds((par * _NT + wid) * 32, 32)])
        plsc.subcore_barrier()
        pltpu.sync_copy(sh_cnt.at[pl.ds(par * (_NT * 32), _NT * 32)], rd)
        sA = zero
        sB = zero
        for tt in range(_NT):
            sA = sA + rd[pl.ds(tt * 32, 16)]
            sB = sB + rd[pl.ds(tt * 32 + 16, 16)]
        cA = _lanesum_u32(sA)
        cB = _lanesum_u32(sB)
        resA = jnp.where(cA >= kA1, resA, midA)
        resB = jnp.where(cB >= kB1, resB, midB)
        return (resA, resB, bit >> jnp.uint32(1))

    resA, resB, _ = lax.fori_loop(
        0, 32, round_body,
        (jnp.uint32(0), jnp.uint32(0), jnp.uint32(0x80000000)))

    def fin_body(i, acc):
        leA, gtA, leB, gtB = acc
        for j in range(_UNROLL):
            off = (i * _UNROLL + j) * 16
            u = ubuf[pl.ds(off, 16)]
            leA = leA + jnp.where(u <= resA, one, zero)
            gtA = jnp.minimum(gtA, jnp.where(u > resA, u, umax_v))
            leB = leB + jnp.where(u <= resB, one, zero)
            gtB = jnp.minimum(gtB, jnp.where(u > resB, u, umax_v))
        return (leA, gtA, leB, gtB)
    leA, gtA, leB, gtB = lax.fori_loop(
        0, _VECS // _UNROLL, fin_body, (zero, umax_v, zero, umax_v))

    fin[pl.ds(0, 16)] = leA
    fin[pl.ds(16, 16)] = gtA
    fin[pl.ds(32, 16)] = leB
    fin[pl.ds(48, 16)] = gtB
    pltpu.sync_copy(fin, sh_fin.at[pl.ds(wid * 64, 64)])
    plsc.subcore_barrier()
    pltpu.sync_copy(sh_fin, rdf)
    sLeA = zero
    sLeB = zero
    mGtA = umax_v
    mGtB = umax_v
    for tt in range(_NT):
        sLeA = sLeA + rdf[pl.ds(tt * 64, 16)]
        mGtA = jnp.minimum(mGtA, rdf[pl.ds(tt * 64 + 16, 16)])
        sLeB = sLeB + rdf[pl.ds(tt * 64 + 32, 16)]
        mGtB = jnp.minimum(mGtB, rdf[pl.ds(tt * 64 + 48, 16)])
    cLeA = _lanesum_u32(sLeA)
    cLeB = _lanesum_u32(sLeB)
    minGtA = _lanemin_u32(mGtA)
    minGtB = _lanemin_u32(mGtB)

    vA1 = jnp.where(cLeA >= jnp.uint32(_KA + 2), resA, minGtA)
    vB1 = jnp.where(cLeB >= jnp.uint32(_KB + 2), resB, minGtB)

    def unmap(uv):
        sgn = uv >> jnp.uint32(31)
        bits = jnp.where(sgn == jnp.uint32(1),
                         uv ^ jnp.uint32(0x80000000), ~uv)
        return lax.bitcast_convert_type(bits, jnp.float32)

    vA0f = unmap(resA)
    vA1f = unmap(vA1)
    vB0f = unmap(resB)
    vB1f = unmap(vB1)

    f32 = jnp.float32
    q_low = vA0f * f32(1.0 - _FRACA) + vA1f * f32(_FRACA)
    q_high = vB0f * f32(1.0 - _FRACB) + vB1f * f32(_FRACB)

    pv = pbuf[pl.ds(0, 16)]
    low_s = pv[0]
    high_s = pv[1]
    new_low = f32(_DECAY) * low_s + f32(1.0 - _DECAY) * q_low
    new_high = f32(_DECAY) * high_s + f32(1.0 - _DECAY) * q_high
    inv_scale = jnp.maximum(f32(_MIN), new_high - new_low)

    lanes = lax.iota(jnp.int32, 16)
    zf = jnp.zeros((16,), f32)
    obuf[pl.ds(0, 16)] = jnp.where(
        lanes == 0, new_low, jnp.where(lanes == 1, inv_scale, zf))

    @pl.when(wid == 0)
    def _():
        pltpu.sync_copy(obuf, out_hbm)


_moments_sc = functools.partial(
    pl.kernel,
    out_type=jax.ShapeDtypeStruct((16,), jnp.float32),
    mesh=_mesh,
    scratch_types=[
        pltpu.VMEM((_CHUNK,), jnp.float32),
        pltpu.VMEM((_CHUNK,), jnp.uint32),
        pltpu.VMEM((32,), jnp.uint32),
        pltpu.VMEM((_NT * 32,), jnp.uint32),
        pltpu.VMEM((64,), jnp.uint32),
        pltpu.VMEM((_NT * 64,), jnp.uint32),
        pltpu.VMEM((16,), jnp.float32),
        pltpu.VMEM((16,), jnp.float32),
        pltpu.VMEM_SHARED((2 * _NT * 32,), jnp.uint32),
        pltpu.VMEM_SHARED((_NT * 64,), jnp.uint32),
    ],
)(_body)


def kernel(x, low, high):
    xf = x.reshape(-1)
    p = jnp.zeros((16,), jnp.float32).at[0].set(low).at[1].set(high)
    out = _moments_sc(xf, p)
    return (out[0], out[1])
